# baseline (speedup 1.0000x reference)
import functools

import jax
import jax.numpy as jnp
from jax import lax
from jax.experimental import pallas as pl
from jax.experimental.pallas import tpu as pltpu
from jax.experimental.pallas import tpu_sc as plsc

_GS = 128
_LANES = 16
_VPG = _GS // _LANES
_CH = 256
_U = 2


def _sort16(v):
    return lax.sort(v, dimension=0)


def _rev(v):
    return lax.rev(v, (0,))


def _permute(v, idx):
    return jnp.take_along_axis(v, idx, axis=0, mode="promise_in_bounds")


def _bitonic16(v, lane):
    for s in (8, 4, 2, 1):
        p = _permute(v, lane ^ s)
        mn = jnp.minimum(v, p)
        mx = jnp.maximum(v, p)
        v = jnp.where((lane & s) != 0, mx, mn)
    return v


_PERM_MASK = {
    1: (False, False),
    2: (True, False, True, False),
    3: (True,) * 8,
}


def _merge(a, b, lane, perm_mask):
    m = len(a)
    c = a + [_rev(b[m - 1 - i]) for i in range(m)]
    stride = m
    while stride >= 1:
        nxt = list(c)
        for base in range(0, 2 * m, 2 * stride):
            for i in range(stride):
                lo, hi = c[base + i], c[base + stride + i]
                nxt[base + i] = jnp.minimum(lo, hi)
                nxt[base + stride + i] = jnp.maximum(lo, hi)
        c = nxt
        stride //= 2
    return [_bitonic16(v, lane) if perm_mask[i] else _sort16(v)
            for i, v in enumerate(c)]


def _sort_group(vs, lane):
    runs = [[_sort16(v)] for v in vs]
    rnd = 0
    while len(runs) > 1:
        rnd += 1
        runs = [_merge(runs[2 * i], runs[2 * i + 1], lane, _PERM_MASK[rnd])
                for i in range(len(runs) // 2)]
    return runs[0]


@functools.lru_cache(maxsize=None)
def _build(rows):
    info = plsc.get_sparse_core_info()
    nc, ns = info.num_cores, info.num_subcores
    nw = nc * ns
    rpw = rows // nw
    ch = min(_CH, rpw)
    mesh = plsc.VectorSubcoreMesh(core_axis_name="c", subcore_axis_name="s")

    nch = rpw // ch

    @functools.partial(
        pl.kernel,
        mesh=mesh,
        out_type=jax.ShapeDtypeStruct((rows, _GS), jnp.float32),
        scratch_types=[
            pltpu.VMEM((ch, _GS), jnp.float32),
            pltpu.VMEM((ch, _GS), jnp.float32),
            pltpu.SemaphoreType.DMA,
            pltpu.SemaphoreType.DMA,
            pltpu.SemaphoreType.DMA,
            pltpu.SemaphoreType.DMA,
        ],
        compiler_params=pltpu.CompilerParams(needs_layout_passes=False),
    )
    def sc_group_sort(x_hbm, out_hbm, b0, b1, si0, si1, so0, so1):
        wid = lax.axis_index("s") * nc + lax.axis_index("c")
        row0 = wid * rpw
        bufs, sin, sout = [b0, b1], [si0, si1], [so0, so1]

        def start_in(c):
            base = row0 + c * ch
            return pltpu.async_copy(
                x_hbm.at[pl.ds(base, ch)], bufs[c % 2], sin[c % 2])

        def start_out(c):
            base = row0 + c * ch
            return pltpu.async_copy(
                bufs[c % 2], out_hbm.at[pl.ds(base, ch)], sout[c % 2])

        in_h = {0: start_in(0)}
        out_h = {}
        for c in range(nch):
            if c + 1 < nch:
                if c - 1 >= 0:
                    out_h[c - 1].wait()
                in_h[c + 1] = start_in(c + 1)
            in_h[c].wait()
            buf = bufs[c % 2]

            lane = lax.iota(jnp.int32, 16)

            @plsc.parallel_loop(0, ch, step=1, unroll=_U)
            def body(g):
                vs = [buf[g, pl.ds(j * _LANES, _LANES)]
                      for j in range(_VPG)]
                sv = _sort_group(vs, lane)
                for j in range(_VPG):
                    buf[g, pl.ds(j * _LANES, _LANES)] = sv[j]

            out_h[c] = start_out(c)
        for c in range(max(0, nch - 2), nch):
            out_h[c].wait()

    return sc_group_sort


def kernel(x):
    b, f = x.shape
    rows = b * f // _GS
    xr = x.reshape(rows, _GS)
    out = _build(rows)(xr)
    return out.reshape(b, f)


# speedup vs baseline: 1.3412x; 1.3412x over previous
import functools

import jax
import jax.numpy as jnp
from jax import lax
from jax.experimental import pallas as pl
from jax.experimental.pallas import tpu as pltpu
from jax.experimental.pallas import tpu_sc as plsc

_GS = 128
_LANES = 16
_VPG = _GS // _LANES
_CH = 256
_U = 4


def _sort16(v):
    return lax.sort(v, dimension=0)


def _rev(v):
    return lax.rev(v, (0,))


def _permute(v, idx):
    return jnp.take_along_axis(v, idx, axis=0, mode="promise_in_bounds")


def _bitonic16(v, lane):
    for s in (8, 4, 2, 1):
        p = _permute(v, lane ^ s)
        mn = jnp.minimum(v, p)
        mx = jnp.maximum(v, p)
        v = jnp.where((lane & s) != 0, mx, mn)
    return v


_PERM_MASK = {
    1: (False, False),
    2: (False, False, False, False),
    3: (False,) * 8,
}


def _merge(a, b, lane, perm_mask):
    m = len(a)
    c = a + [_rev(b[m - 1 - i]) for i in range(m)]
    stride = m
    while stride >= 1:
        nxt = list(c)
        for base in range(0, 2 * m, 2 * stride):
            for i in range(stride):
                lo, hi = c[base + i], c[base + stride + i]
                nxt[base + i] = jnp.minimum(lo, hi)
                nxt[base + stride + i] = jnp.maximum(lo, hi)
        c = nxt
        stride //= 2
    return [_bitonic16(v, lane) if perm_mask[i] else _sort16(v)
            for i, v in enumerate(c)]


def _sort_group(vs, lane):
    runs = [[_sort16(v)] for v in vs]
    rnd = 0
    while len(runs) > 1:
        rnd += 1
        runs = [_merge(runs[2 * i], runs[2 * i + 1], lane, _PERM_MASK[rnd])
                for i in range(len(runs) // 2)]
    return runs[0]


@functools.lru_cache(maxsize=None)
def _build(rows):
    info = plsc.get_sparse_core_info()
    nc, ns = info.num_cores, info.num_subcores
    nw = nc * ns
    rpw = rows // nw
    ch = min(_CH, rpw)
    mesh = plsc.VectorSubcoreMesh(core_axis_name="c", subcore_axis_name="s")

    nch = rpw // ch

    @functools.partial(
        pl.kernel,
        mesh=mesh,
        out_type=jax.ShapeDtypeStruct((rows, _GS), jnp.float32),
        scratch_types=[
            pltpu.VMEM((ch, _GS), jnp.float32),
            pltpu.VMEM((ch, _GS), jnp.float32),
            pltpu.SemaphoreType.DMA,
            pltpu.SemaphoreType.DMA,
            pltpu.SemaphoreType.DMA,
            pltpu.SemaphoreType.DMA,
        ],
        compiler_params=pltpu.CompilerParams(needs_layout_passes=False),
    )
    def sc_group_sort(x_hbm, out_hbm, b0, b1, si0, si1, so0, so1):
        wid = lax.axis_index("s") * nc + lax.axis_index("c")
        row0 = wid * rpw
        bufs, sin, sout = [b0, b1], [si0, si1], [so0, so1]

        def start_in(c):
            base = row0 + c * ch
            return pltpu.async_copy(
                x_hbm.at[pl.ds(base, ch)], bufs[c % 2], sin[c % 2])

        def start_out(c):
            base = row0 + c * ch
            return pltpu.async_copy(
                bufs[c % 2], out_hbm.at[pl.ds(base, ch)], sout[c % 2])

        in_h = {0: start_in(0)}
        out_h = {}
        for c in range(nch):
            if c + 1 < nch:
                if c - 1 >= 0:
                    out_h[c - 1].wait()
                in_h[c + 1] = start_in(c + 1)
            in_h[c].wait()
            buf = bufs[c % 2]

            lane = lax.iota(jnp.int32, 16)

            @plsc.parallel_loop(0, ch, step=1, unroll=_U)
            def body(g):
                vs = [buf[g, pl.ds(j * _LANES, _LANES)]
                      for j in range(_VPG)]
                sv = _sort_group(vs, lane)
                for j in range(_VPG):
                    buf[g, pl.ds(j * _LANES, _LANES)] = sv[j]

            out_h[c] = start_out(c)
        for c in range(max(0, nch - 2), nch):
            out_h[c].wait()

    return sc_group_sort


def kernel(x):
    b, f = x.shape
    rows = b * f // _GS
    xr = x.reshape(rows, _GS)
    out = _build(rows)(xr)
    return out.reshape(b, f)
